# baseline (speedup 1.0000x reference)
import functools

import jax
import jax.numpy as jnp
from jax import lax
from jax.experimental import pallas as pl
from jax.experimental.pallas import tpu as pltpu
from jax.experimental.pallas import tpu_sc as plsc

B, S, D = 4, 4096, 2048
E = 16
CHUNK = 128
H = 1024
H2 = 512
TAU = 0.7
NC = S // CHUNK

BLK = 2048
CPB = BLK // CHUNK
NT = B * S
NG = NT // BLK
NCH = NT // CHUNK
CPT = NCH // 32


def _mlp_body(x_ref, w1_ref, b1_ref, w2_ref, b2_ref, w3_ref, b3_ref,
              cl_ref, ent_ref):
    x = x_ref[...]
    h = jnp.dot(x, w1_ref[...], preferred_element_type=jnp.float32)
    h = jnp.maximum(h + b1_ref[...], 0.0)
    h2 = jnp.dot(h, w2_ref[...], preferred_element_type=jnp.float32)
    h2 = jnp.maximum(h2 + b2_ref[...], 0.0)
    hm = jnp.mean(h2.reshape(CPB, CHUNK, H2), axis=1)
    logits = jnp.dot(hm, w3_ref[...], preferred_element_type=jnp.float32)
    logits = logits + b3_ref[...]
    cl_ref[0] = logits
    m = jnp.max(logits, axis=-1, keepdims=True)
    ex = jnp.exp(logits - m)
    p = ex / jnp.sum(ex, axis=-1, keepdims=True)
    ent = -(p * jnp.log(p + 1e-8)).sum().reshape(1, 1)

    @pl.when(pl.program_id(0) == 0)
    def _():
        ent_ref[...] = jnp.zeros((1, 1), jnp.float32)

    ent_ref[...] += ent


def _router_mlp(x2, W1, b1, W2, b2, W3, b3):
    cl, ent = pl.pallas_call(
        _mlp_body,
        grid=(NG,),
        in_specs=[
            pl.BlockSpec((BLK, D), lambda i: (i, 0)),
            pl.BlockSpec((D, H), lambda i: (0, 0)),
            pl.BlockSpec((1, H), lambda i: (0, 0)),
            pl.BlockSpec((H, H2), lambda i: (0, 0)),
            pl.BlockSpec((1, H2), lambda i: (0, 0)),
            pl.BlockSpec((H2, E), lambda i: (0, 0)),
            pl.BlockSpec((1, E), lambda i: (0, 0)),
        ],
        out_specs=[
            pl.BlockSpec((1, CPB, E), lambda i: (i, 0, 0)),
            pl.BlockSpec((1, 1), lambda i: (0, 0)),
        ],
        out_shape=[
            jax.ShapeDtypeStruct((NG, CPB, E), jnp.float32),
            jax.ShapeDtypeStruct((1, 1), jnp.float32),
        ],
    )(x2, W1, b1.reshape(1, H), W2, b2.reshape(1, H2), W3, b3.reshape(1, E))
    return cl.reshape(NCH, E), ent


_SC_OPTS = dict(
    mesh=plsc.VectorSubcoreMesh(core_axis_name="c", subcore_axis_name="s"),
    compiler_params=pltpu.CompilerParams(needs_layout_passes=False),
    out_type=[
        jax.ShapeDtypeStruct((NT, E), jnp.float32),
        jax.ShapeDtypeStruct((NCH,), jnp.int32),
        jax.ShapeDtypeStruct((E,), jnp.float32),
        jax.ShapeDtypeStruct((E,), jnp.float32),
    ],
    scratch_types=[
        pltpu.VMEM((NCH, E), jnp.float32),
        pltpu.VMEM((CPT * CHUNK, E), jnp.float32),
        pltpu.VMEM((NCH,), jnp.int32),
        pltpu.VMEM((E,), jnp.float32),
        pltpu.VMEM((E,), jnp.float32),
    ],
)


def _sc_stage2_body(cl_hbm, rw_hbm, ei_hbm, util_hbm, misc_hbm,
                    cl_v, rw_v, ei_v, util_v, misc_v):
    wid = lax.axis_index("s") * 2 + lax.axis_index("c")
    b = wid // 8
    pltpu.sync_copy(cl_hbm.at[pl.ds(b * NC, NC)], cl_v.at[pl.ds(0, NC)])
    iota = lax.iota(jnp.int32, 16)

    def make_step(base):
        def step(i, carry):
            prev_e, e0, e1, fl, cnt = carry
            li = cl_v[base + i]
            top_val = jnp.max(li)
            top = plsc.all_reduce_ffs(li == top_val)
            prv_val = li.at[prev_e].get(mode="promise_in_bounds")
            first = i == 0
            switch = (top_val - prv_val) > TAU
            new_e = jnp.where(jnp.logical_or(first, switch), top, prev_e)
            fl = fl + jnp.where(first, 0.0, switch.astype(jnp.float32))
            cnt = cnt + (iota == new_e).astype(jnp.float32)
            e0 = jnp.where(iota == i, new_e, e0)
            e1 = jnp.where(iota == i - 16, new_e, e1)
            return new_e, e0, e1, fl, cnt
        return step

    z16 = jnp.zeros((16,), jnp.int32)
    z16f = jnp.zeros((16,), jnp.float32)

    _, e0, e1, flips, counts = lax.fori_loop(
        0, NC, make_step(0), (z16, z16, z16, z16f, z16f))
    ei_v[pl.ds(0, 16)] = e0
    ei_v[pl.ds(16, 16)] = e1

    my_e = plsc.load_gather(ei_v, [(wid % 8) * CPT + lax.rem(iota, CPT)])
    onehots = [(iota == jnp.max(jnp.where(iota == j, my_e, jnp.int32(-1))))
               .astype(jnp.float32) for j in range(CPT)]

    def store_body(t, carry):
        for j in range(CPT):
            rw_v[j * CHUNK + t] = onehots[j]
        return carry

    lax.fori_loop(0, CHUNK, store_body, 0)
    pltpu.sync_copy(rw_v, rw_hbm.at[pl.ds(wid * (CPT * CHUNK), CPT * CHUNK)])

    @pl.when(wid == 0)
    def _():
        pltpu.sync_copy(cl_hbm, cl_v)
        fl_cnt = (flips, counts)
        for bb in range(1, B):
            _, e0b, e1b, fls, cnts = lax.fori_loop(
                0, NC, make_step(bb * NC), (z16, z16, z16, *fl_cnt))
            ei_v[pl.ds(bb * NC, 16)] = e0b
            ei_v[pl.ds(bb * NC + 16, 16)] = e1b
            fl_cnt = (fls, cnts)
        fl_tot, cnt_tot = fl_cnt
        pltpu.sync_copy(ei_v, ei_hbm)
        util = cnt_tot * (1.0 / NCH)
        util_v[...] = util
        pltpu.sync_copy(util_v, util_hbm)
        ss = jnp.sum(util * util) * jnp.ones((16,), jnp.float32)
        y = 0.5 * (1.0 + ss)
        for _ in range(6):
            y = 0.5 * (y + ss / y)
        fr = fl_tot * (1.0 / (B * (NC - 1)))
        misc = jnp.where(iota == 0, fr, 0.0)
        misc = jnp.where(iota == 1, y, misc)
        misc_v[...] = misc
        pltpu.sync_copy(misc_v, misc_hbm)


_sc_stage2 = pl.kernel(**_SC_OPTS)(_sc_stage2_body)


def kernel(x, prev_expert_indices, W1, b1, W2, b2, W3, b3):
    x2 = x.reshape(NT, D)
    cl_flat, ent_sum = _router_mlp(x2, W1, b1, W2, b2, W3, b3)
    rw_flat, ei_flat, utilization, misc = _sc_stage2(cl_flat)

    routing_weights = rw_flat.reshape(B, S, E)
    expert_indices = ei_flat.reshape(B, NC)
    chunk_logits = cl_flat.reshape(B, NC, E)
    gate_entropy = ent_sum[0, 0] * (1.0 / NCH)
    flip_rate = misc[0]
    routing_concentration = misc[1]

    return (routing_weights, expert_indices, chunk_logits,
            gate_entropy, utilization, flip_rate, routing_concentration)


# speedup vs baseline: 1.0022x; 1.0022x over previous
import functools

import jax
import jax.numpy as jnp
from jax import lax
from jax.experimental import pallas as pl
from jax.experimental.pallas import tpu as pltpu
from jax.experimental.pallas import tpu_sc as plsc

B, S, D = 4, 4096, 2048
E = 16
CHUNK = 128
H = 1024
H2 = 512
TAU = 0.7
NC = S // CHUNK

BLK = 2048
CPB = BLK // CHUNK
NT = B * S
NG = NT // BLK
NCH = NT // CHUNK
NTILES = 32
CPT = NCH // NTILES
TPB = NTILES // B


def _mlp_body(x_ref, w1_ref, b1_ref, w2_ref, b2_ref, w3_ref, b3_ref,
              cl_ref, ent_ref):
    x = x_ref[...]
    h = jnp.dot(x, w1_ref[...], preferred_element_type=jnp.float32)
    h = jnp.maximum(h + b1_ref[...], 0.0)
    h2 = jnp.dot(h, w2_ref[...], preferred_element_type=jnp.float32)
    h2 = jnp.maximum(h2 + b2_ref[...], 0.0)
    hm = jnp.mean(h2.reshape(CPB, CHUNK, H2), axis=1)
    logits = jnp.dot(hm, w3_ref[...], preferred_element_type=jnp.float32)
    logits = logits + b3_ref[...]
    cl_ref[0] = logits
    m = jnp.max(logits, axis=-1, keepdims=True)
    ex = jnp.exp(logits - m)
    p = ex / jnp.sum(ex, axis=-1, keepdims=True)
    ent = -(p * jnp.log(p + 1e-8)).sum().reshape(1, 1)

    @pl.when(pl.program_id(0) == 0)
    def _():
        ent_ref[...] = jnp.zeros((1, 1), jnp.float32)

    ent_ref[...] += ent


def _router_mlp(x2, W1, b1, W2, b2, W3, b3):
    cl, ent = pl.pallas_call(
        _mlp_body,
        grid=(NG,),
        in_specs=[
            pl.BlockSpec((BLK, D), lambda i: (i, 0)),
            pl.BlockSpec((D, H), lambda i: (0, 0)),
            pl.BlockSpec((1, H), lambda i: (0, 0)),
            pl.BlockSpec((H, H2), lambda i: (0, 0)),
            pl.BlockSpec((1, H2), lambda i: (0, 0)),
            pl.BlockSpec((H2, E), lambda i: (0, 0)),
            pl.BlockSpec((1, E), lambda i: (0, 0)),
        ],
        out_specs=[
            pl.BlockSpec((1, CPB, E), lambda i: (i, 0, 0)),
            pl.BlockSpec((1, 1), lambda i: (0, 0)),
        ],
        out_shape=[
            jax.ShapeDtypeStruct((NG, CPB, E), jnp.float32),
            jax.ShapeDtypeStruct((1, 1), jnp.float32),
        ],
    )(x2, W1, b1.reshape(1, H), W2, b2.reshape(1, H2), W3, b3.reshape(1, E))
    return cl.reshape(NCH, E), ent


_SC_OPTS = dict(
    mesh=plsc.VectorSubcoreMesh(core_axis_name="c", subcore_axis_name="s"),
    compiler_params=pltpu.CompilerParams(needs_layout_passes=False),
    out_type=[
        jax.ShapeDtypeStruct((NT, E), jnp.float32),
        jax.ShapeDtypeStruct((NCH,), jnp.int32),
        jax.ShapeDtypeStruct((E,), jnp.float32),
        jax.ShapeDtypeStruct((E,), jnp.float32),
    ],
    scratch_types=[
        pltpu.VMEM((NCH, E), jnp.float32),
        pltpu.VMEM((CPT * CHUNK, E), jnp.float32),
        pltpu.VMEM((NCH,), jnp.int32),
        pltpu.VMEM((E,), jnp.float32),
        pltpu.VMEM((E,), jnp.float32),
    ],
)


def _sc_stage2_body(cl_hbm, rw_hbm, ei_hbm, util_hbm, misc_hbm,
                    cl_v, rw_v, ei_v, util_v, misc_v):
    wid = lax.axis_index("s") * 2 + lax.axis_index("c")
    b = wid // TPB
    pltpu.sync_copy(cl_hbm.at[pl.ds(b * NC, NC)], cl_v.at[pl.ds(0, NC)])
    iota = lax.iota(jnp.int32, 16)

    def make_step(base):
        def step(i, carry):
            prev_e, e0, e1, fl, cnt = carry
            li = cl_v[base + i]
            top_val = jnp.max(li)
            top = plsc.all_reduce_ffs(li == top_val)
            prv_val = li.at[prev_e].get(mode="promise_in_bounds")
            first = i == 0
            switch = (top_val - prv_val) > TAU
            new_e = jnp.where(jnp.logical_or(first, switch), top, prev_e)
            fl = fl + jnp.where(first, 0.0, switch.astype(jnp.float32))
            cnt = cnt + (iota == new_e).astype(jnp.float32)
            e0 = jnp.where(iota == i, new_e, e0)
            e1 = jnp.where(iota == i - 16, new_e, e1)
            return new_e, e0, e1, fl, cnt
        return step

    z16 = jnp.zeros((16,), jnp.int32)
    z16f = jnp.zeros((16,), jnp.float32)

    _, e0, e1, flips, counts = lax.fori_loop(
        0, NC, make_step(0), (z16, z16, z16, z16f, z16f))
    ei_v[pl.ds(0, 16)] = e0
    ei_v[pl.ds(16, 16)] = e1

    my_e = plsc.load_gather(ei_v, [(wid % TPB) * CPT + lax.rem(iota, CPT)])
    onehots = [(iota == jnp.max(jnp.where(iota == j, my_e, jnp.int32(-1))))
               .astype(jnp.float32) for j in range(CPT)]

    def store_body(t, carry):
        for j in range(CPT):
            rw_v[j * CHUNK + t] = onehots[j]
        return carry

    lax.fori_loop(0, CHUNK, store_body, 0)
    pltpu.sync_copy(rw_v, rw_hbm.at[pl.ds(wid * (CPT * CHUNK), CPT * CHUNK)])

    @pl.when(wid == 0)
    def _():
        pltpu.sync_copy(cl_hbm, cl_v)
        fl_cnt = (flips, counts)
        for bb in range(1, B):
            _, e0b, e1b, fls, cnts = lax.fori_loop(
                0, NC, make_step(bb * NC), (z16, z16, z16, *fl_cnt))
            ei_v[pl.ds(bb * NC, 16)] = e0b
            ei_v[pl.ds(bb * NC + 16, 16)] = e1b
            fl_cnt = (fls, cnts)
        fl_tot, cnt_tot = fl_cnt
        pltpu.sync_copy(ei_v, ei_hbm)
        util = cnt_tot * (1.0 / NCH)
        util_v[...] = util
        pltpu.sync_copy(util_v, util_hbm)
        ss = jnp.sum(util * util) * jnp.ones((16,), jnp.float32)
        y = 0.5 * (1.0 + ss)
        for _ in range(6):
            y = 0.5 * (y + ss / y)
        fr = fl_tot * (1.0 / (B * (NC - 1)))
        misc = jnp.where(iota == 0, fr, 0.0)
        misc = jnp.where(iota == 1, y, misc)
        misc_v[...] = misc
        pltpu.sync_copy(misc_v, misc_hbm)


_sc_stage2 = pl.kernel(**_SC_OPTS)(_sc_stage2_body)


def kernel(x, prev_expert_indices, W1, b1, W2, b2, W3, b3):
    x2 = x.reshape(NT, D)
    cl_flat, ent_sum = _router_mlp(x2, W1, b1, W2, b2, W3, b3)
    rw_flat, ei_flat, utilization, misc = _sc_stage2(cl_flat)

    routing_weights = rw_flat.reshape(B, S, E)
    expert_indices = ei_flat.reshape(B, NC)
    chunk_logits = cl_flat.reshape(B, NC, E)
    gate_entropy = ent_sum[0, 0] * (1.0 / NCH)
    flip_rate = misc[0]
    routing_concentration = misc[1]

    return (routing_weights, expert_indices, chunk_logits,
            gate_entropy, utilization, flip_rate, routing_concentration)
